# baseline (speedup 1.0000x reference)
import jax
import jax.numpy as jnp
from jax import lax
from jax.experimental import pallas as pl
from jax.experimental.pallas import tpu as pltpu

_NB = 8


def _wband_tiles(wk, W, C):
    WC = W * C
    wi = jnp.arange(W)[:, None]
    wo = jnp.arange(W)[None, :]
    dw = wi - wo + 1
    okw = ((dw >= 0) & (dw <= 2)).astype(wk.dtype)
    q = wk[:, :, jnp.clip(dw, 0, 2)]
    q = q * okw[None, None, :, :, None, None]
    q = jnp.transpose(q, (0, 1, 2, 4, 3, 5)).reshape(3, 3, WC, WC)
    qz = jnp.concatenate([q, jnp.zeros((3, 1, WC, WC), wk.dtype)], axis=1)
    return qz.astype(jnp.bfloat16)


def _shifted(a):
    NB, D, L = a.shape
    z = jnp.zeros((NB, 1, L), a.dtype)
    up = jnp.concatenate([z, a[:, :-1]], axis=1)
    dn = jnp.concatenate([a[:, 1:], z], axis=1)
    return up, dn


def _chan_pattern(C, L):
    l = lax.broadcasted_iota(jnp.int32, (C, L), 1)
    c = lax.broadcasted_iota(jnp.int32, (C, L), 0)
    return (l % C == c).astype(jnp.float32)


def kernel(x, w1_oi, w2_oi, w1x1_oi, w1, w2, w1x1,
           b1, b2, b1x1, g1, be1, g2, be2):
    N, C, D, H, W = x.shape
    HW = H * W
    L = HW * C
    WC = W * C
    NG = H // 2
    GW = 2 * WC
    KH = 4
    NB = min(_NB, N)
    G = N // NB
    M = NB * D
    count = float(N * D * HW)

    x4 = x.astype(jnp.bfloat16).reshape(N, C, D, HW)

    qa = _wband_tiles(w1, W, C)
    qb = _wband_tiles(w2, W, C)
    b256 = jnp.kron(jnp.eye(2 * W, dtype=w1x1.dtype), w1x1).astype(jnp.bfloat16)
    vecs = jnp.stack([b1, b2, b1x1, g1, be1, g2, be2], axis=0)

    g_start = [min(max((2 * k - 1) * WC, 0), L - KH * WC) for k in range(NG)]

    def body(x4_ref, qa_ref, qb_ref, b256_ref, vecs_ref, o_ref,
             xb_scr, y1_scr, y2_scr, wa_scr, wb_scr,
             pp_scr, pt_scr, vt_scr, st_scr, bn_scr):
        s = pl.program_id(0)
        g = pl.program_id(1)

        @pl.when(jnp.logical_and(s == 0, g == 0))
        def _prep():
            P = _chan_pattern(C, L)
            vt_scr[0:7, :] = jnp.dot(vecs_ref[...], P,
                                     preferred_element_type=jnp.float32)
            st_scr[...] = jnp.zeros_like(st_scr)
            r = lax.broadcasted_iota(jnp.int32, (L, L), 0)
            c_ = lax.broadcasted_iota(jnp.int32, (L, L), 1)
            pp_scr[...] = (((r // HW) == (c_ % C)) &
                           ((r % HW) == (c_ // C))).astype(jnp.bfloat16)
            pt_scr[...] = (((r % C) == (c_ // HW)) &
                           ((r // C) == (c_ % HW))).astype(jnp.bfloat16)
            zt = jnp.zeros((WC, WC), jnp.bfloat16)
            for k in range(NG):
                hi0 = g_start[k] // WC
                for kd in range(3):
                    for hr in range(KH):
                        hi = hi0 + hr
                        ta, tb = [], []
                        for ho in (2 * k, 2 * k + 1):
                            dh = hi - ho + 1
                            if 0 <= dh <= 2:
                                ta.append(qa_ref[kd, dh])
                                tb.append(qb_ref[kd, dh])
                            else:
                                ta.append(zt)
                                tb.append(zt)
                        rr = (kd * KH + hr) * WC
                        wa_scr[k, rr:rr + WC, :] = jnp.concatenate(ta, axis=1)
                        wb_scr[k, rr:rr + WC, :] = jnp.concatenate(tb, axis=1)

        def _conv_groups(a, w_scr):
            up, dn = _shifted(a)
            a2d = a.reshape(M, L)
            up2d = up.reshape(M, L)
            dn2d = dn.reshape(M, L)
            KW = KH * WC
            outs = []
            for k in range(NG):
                st = g_start[k]
                acc = jnp.dot(up2d[:, st:st + KW], w_scr[k, 0:KW],
                              preferred_element_type=jnp.float32)
                acc += jnp.dot(a2d[:, st:st + KW], w_scr[k, KW:2 * KW],
                               preferred_element_type=jnp.float32)
                acc += jnp.dot(dn2d[:, st:st + KW], w_scr[k, 2 * KW:3 * KW],
                               preferred_element_type=jnp.float32)
                outs.append(acc)
            return outs

        def _stage_store(accs, b_row, s_row, y_scr):
            for k in range(NG):
                acc = accs[k] + vt_scr[b_row:b_row + 1, k * GW:(k + 1) * GW]
                st_scr[s_row:s_row + 1, k * GW:(k + 1) * GW] += \
                    jnp.sum(acc, axis=0, keepdims=True)
                st_scr[s_row + 1:s_row + 2, k * GW:(k + 1) * GW] += \
                    jnp.sum(acc * acc, axis=0, keepdims=True)
                y_scr[pl.ds(g * NB, NB), :, k * GW:(k + 1) * GW] = \
                    acc.reshape(NB, D, GW).astype(jnp.bfloat16)

        @pl.when(s == 0)
        def _stage1():
            v = x4_ref[...]
            cat = jnp.concatenate([v[:, c] for c in range(C)],
                                  axis=-1)
            xb = jnp.dot(cat.reshape(M, L), pp_scr[...],
                         preferred_element_type=jnp.float32).astype(jnp.bfloat16)
            xb3 = xb.reshape(NB, D, L)
            xb_scr[pl.ds(g * NB, NB)] = xb3
            _stage_store(_conv_groups(xb3, wa_scr), 0, 0, y1_scr)

        def _fold(s_row, gam_row, out_row):
            P = _chan_pattern(C, L)
            sq = jnp.dot(st_scr[s_row:s_row + 2], P.T,
                         preferred_element_type=jnp.float32)
            mean = sq[0:1] / count
            var = sq[1:2] / count - mean * mean
            scale = vecs_ref[gam_row:gam_row + 1] * lax.rsqrt(var + 1e-5)
            shift = vecs_ref[gam_row + 1:gam_row + 2] - mean * scale
            bn_scr[out_row:out_row + 2] = jnp.dot(
                jnp.concatenate([scale, shift], axis=0), P,
                preferred_element_type=jnp.float32)

        @pl.when(jnp.logical_and(s == 1, g == 0))
        def _fold1():
            _fold(0, 3, 0)

        @pl.when(s == 1)
        def _stage2():
            y1 = y1_scr[pl.ds(g * NB, NB)]
            sc = bn_scr[0:1].astype(jnp.bfloat16)
            sh = bn_scr[1:2].astype(jnp.bfloat16)
            a = jnp.maximum(y1 * sc + sh, jnp.bfloat16(0.0))
            _stage_store(_conv_groups(a, wb_scr), 1, 2, y2_scr)

        @pl.when(jnp.logical_and(s == 2, g == 0))
        def _fold2():
            _fold(2, 5, 2)

        @pl.when(s == 2)
        def _epilogue():
            y2 = y2_scr[pl.ds(g * NB, NB)].astype(jnp.float32)
            a2 = jnp.maximum(y2 * bn_scr[2:3] + bn_scr[3:4], 0.0)
            xf = xb_scr[pl.ds(g * NB, NB)].reshape(M, L)
            res = jnp.concatenate(
                [jnp.dot(xf[:, k * GW:(k + 1) * GW], b256_ref[...],
                         preferred_element_type=jnp.float32)
                 for k in range(L // GW)], axis=1)
            res = res + vt_scr[2:3]
            val = jnp.maximum(a2 + res.reshape(NB, D, L), 0.0)
            valp = jnp.dot(val.reshape(M, L).astype(jnp.bfloat16), pt_scr[...],
                           preferred_element_type=jnp.float32)
            valp3 = valp.reshape(NB, D, L)
            for c in range(C):
                o_ref[:, c] = valp3[:, :, c * HW:(c + 1) * HW]

    outf = pl.pallas_call(
        body,
        out_shape=jax.ShapeDtypeStruct((N, C, D, HW), jnp.float32),
        grid=(3, G),
        in_specs=[
            pl.BlockSpec((NB, C, D, HW),
                         lambda s, g: (jnp.where(s == 0, g, 0), 0, 0, 0)),
            pl.BlockSpec((3, 4, WC, WC), lambda s, g: (0, 0, 0, 0)),
            pl.BlockSpec((3, 4, WC, WC), lambda s, g: (0, 0, 0, 0)),
            pl.BlockSpec((GW, GW), lambda s, g: (0, 0)),
            pl.BlockSpec((7, C), lambda s, g: (0, 0)),
        ],
        out_specs=pl.BlockSpec((NB, C, D, HW),
                               lambda s, g: (jnp.where(s == 2, g, 0), 0, 0, 0)),
        scratch_shapes=[
            pltpu.VMEM((N, D, L), jnp.bfloat16),
            pltpu.VMEM((N, D, L), jnp.bfloat16),
            pltpu.VMEM((N, D, L), jnp.bfloat16),
            pltpu.VMEM((NG, 3 * KH * WC, GW), jnp.bfloat16),
            pltpu.VMEM((NG, 3 * KH * WC, GW), jnp.bfloat16),
            pltpu.VMEM((L, L), jnp.bfloat16),
            pltpu.VMEM((L, L), jnp.bfloat16),
            pltpu.VMEM((8, L), jnp.float32),
            pltpu.VMEM((4, L), jnp.float32),
            pltpu.VMEM((4, L), jnp.float32),
        ],
        compiler_params=pltpu.CompilerParams(
            dimension_semantics=("arbitrary", "arbitrary"),
            vmem_limit_bytes=60 * 1024 * 1024,
        ),
    )(x4, qa, qb, b256, vecs)

    return outf.reshape(N, C, D, H, W)


# speedup vs baseline: 1.2384x; 1.2384x over previous
import jax
import jax.numpy as jnp
from jax import lax
from jax.experimental import pallas as pl
from jax.experimental.pallas import tpu as pltpu

_NB = 8


def _chan_pattern(C, L):
    l = lax.broadcasted_iota(jnp.int32, (C, L), 1)
    c = lax.broadcasted_iota(jnp.int32, (C, L), 0)
    return (l % C == c).astype(jnp.float32)


def _shifted(a):
    NB, D, L = a.shape
    z = jnp.zeros((NB, 1, L), a.dtype)
    up = jnp.concatenate([z, a[:, :-1]], axis=1)
    dn = jnp.concatenate([a[:, 1:], z], axis=1)
    return up, dn


def kernel(x, w1_oi, w2_oi, w1x1_oi, w1, w2, w1x1,
           b1, b2, b1x1, g1, be1, g2, be2):
    N, C, D, H, W = x.shape
    HW = H * W
    L = HW * C
    WC = W * C
    NG = H // 2
    GW = 2 * WC
    KH = 4
    NB = min(_NB, N)
    G = N // NB
    M = NB * D
    count = float(N * D * HW)

    x4 = x.reshape(N, C, D, HW)
    w1r = w1.reshape(27, C, C)
    w2r = w2.reshape(27, C, C)
    vecs = [v.reshape(1, C) for v in (b1, b2, b1x1, g1, be1, g2, be2)]

    g_start = [min(max((2 * k - 1) * WC, 0), L - KH * WC) for k in range(NG)]

    def body(x4_ref, w1_ref, w2_ref, wr_ref,
             b1_ref, b2_ref, br_ref, g1_ref, be1_ref, g2_ref, be2_ref,
             o_ref,
             xb_scr, y1_scr, y2_scr, wa_scr, wb_scr, br256_scr,
             pp_scr, pt_scr, vt_scr, st_scr, bn_scr):
        s = pl.program_id(0)
        g = pl.program_id(1)

        @pl.when(jnp.logical_and(s == 0, g == 0))
        def _prep():
            P = _chan_pattern(C, L)
            vt_scr[0:3, :] = jnp.dot(
                jnp.concatenate([b1_ref[...], b2_ref[...], br_ref[...]],
                                axis=0), P, preferred_element_type=jnp.float32)
            st_scr[...] = jnp.zeros_like(st_scr)
            r = lax.broadcasted_iota(jnp.int32, (L, L), 0)
            c_ = lax.broadcasted_iota(jnp.int32, (L, L), 1)
            pp_scr[...] = (((r // HW) == (c_ % C)) &
                           ((r % HW) == (c_ // C))).astype(jnp.bfloat16)
            pt_scr[...] = (((r % C) == (c_ // HW)) &
                           ((r // C) == (c_ % HW))).astype(jnp.bfloat16)
            rc = lax.broadcasted_iota(jnp.int32, (WC, C), 0) % C
            cc = lax.broadcasted_iota(jnp.int32, (WC, C), 1)
            R = (rc == cc).astype(jnp.float32)
            u = lax.broadcasted_iota(jnp.int32, (WC, WC), 0) // C
            v = lax.broadcasted_iota(jnp.int32, (WC, WC), 1) // C
            dwm = [(u - v + 1 == dw).astype(jnp.float32) for dw in range(3)]

            def band_tile(w_ref, kd, dh):
                t = jnp.zeros((WC, WC), jnp.float32)
                for dw in range(3):
                    slab = jnp.dot(
                        jnp.dot(R, w_ref[kd * 9 + dh * 3 + dw],
                                preferred_element_type=jnp.float32), R.T,
                        preferred_element_type=jnp.float32)
                    t = t + dwm[dw] * slab
                return t.astype(jnp.bfloat16)

            qa = [[band_tile(w1_ref, kd, dh) for dh in range(3)]
                  for kd in range(3)]
            qb = [[band_tile(w2_ref, kd, dh) for dh in range(3)]
                  for kd in range(3)]
            r2c = lax.broadcasted_iota(jnp.int32, (GW, C), 0) % C
            c2c = lax.broadcasted_iota(jnp.int32, (GW, C), 1)
            R2 = (r2c == c2c).astype(jnp.float32)
            u2 = lax.broadcasted_iota(jnp.int32, (GW, GW), 0) // C
            v2 = lax.broadcasted_iota(jnp.int32, (GW, GW), 1) // C
            bd = (u2 == v2).astype(jnp.float32)
            br256_scr[...] = (bd * jnp.dot(
                jnp.dot(R2, wr_ref[...],
                        preferred_element_type=jnp.float32), R2.T,
                preferred_element_type=jnp.float32)).astype(jnp.bfloat16)
            zt = jnp.zeros((WC, WC), jnp.bfloat16)
            for k in range(NG):
                hi0 = g_start[k] // WC
                for kd in range(3):
                    for hr in range(KH):
                        hi = hi0 + hr
                        ta, tb = [], []
                        for ho in (2 * k, 2 * k + 1):
                            dh = hi - ho + 1
                            if 0 <= dh <= 2:
                                ta.append(qa[kd][dh])
                                tb.append(qb[kd][dh])
                            else:
                                ta.append(zt)
                                tb.append(zt)
                        rr = (kd * KH + hr) * WC
                        wa_scr[k, rr:rr + WC, :] = jnp.concatenate(ta, axis=1)
                        wb_scr[k, rr:rr + WC, :] = jnp.concatenate(tb, axis=1)

        def _conv_groups(a, w_scr):
            up, dn = _shifted(a)
            a2d = a.reshape(M, L)
            up2d = up.reshape(M, L)
            dn2d = dn.reshape(M, L)
            KW = KH * WC
            outs = []
            for k in range(NG):
                st = g_start[k]
                acc = jnp.dot(up2d[:, st:st + KW], w_scr[k, 0:KW],
                              preferred_element_type=jnp.float32)
                acc += jnp.dot(a2d[:, st:st + KW], w_scr[k, KW:2 * KW],
                               preferred_element_type=jnp.float32)
                acc += jnp.dot(dn2d[:, st:st + KW], w_scr[k, 2 * KW:3 * KW],
                               preferred_element_type=jnp.float32)
                outs.append(acc)
            return outs

        def _stage_store(accs, b_row, s_row, y_scr):
            for k in range(NG):
                acc = accs[k] + vt_scr[b_row:b_row + 1, k * GW:(k + 1) * GW]
                st_scr[s_row:s_row + 1, k * GW:(k + 1) * GW] += \
                    jnp.sum(acc, axis=0, keepdims=True)
                st_scr[s_row + 1:s_row + 2, k * GW:(k + 1) * GW] += \
                    jnp.sum(acc * acc, axis=0, keepdims=True)
                y_scr[pl.ds(g * NB, NB), :, k * GW:(k + 1) * GW] = \
                    acc.reshape(NB, D, GW).astype(jnp.bfloat16)

        @pl.when(s == 0)
        def _stage1():
            v = x4_ref[...].astype(jnp.bfloat16)
            cat = jnp.concatenate([v[:, c] for c in range(C)],
                                  axis=-1)
            xb = jnp.dot(cat.reshape(M, L), pp_scr[...],
                         preferred_element_type=jnp.float32).astype(jnp.bfloat16)
            xb3 = xb.reshape(NB, D, L)
            xb_scr[pl.ds(g * NB, NB)] = xb3
            _stage_store(_conv_groups(xb3, wa_scr), 0, 0, y1_scr)

        def _fold(s_row, gam_ref, bet_ref, out_row):
            P = _chan_pattern(C, L)
            sq = jnp.dot(st_scr[s_row:s_row + 2], P.T,
                         preferred_element_type=jnp.float32)
            mean = sq[0:1] / count
            var = sq[1:2] / count - mean * mean
            scale = gam_ref[...] * lax.rsqrt(var + 1e-5)
            shift = bet_ref[...] - mean * scale
            bn_scr[out_row:out_row + 2] = jnp.dot(
                jnp.concatenate([scale, shift], axis=0), P,
                preferred_element_type=jnp.float32)

        @pl.when(jnp.logical_and(s == 1, g == 0))
        def _fold1():
            _fold(0, g1_ref, be1_ref, 0)

        @pl.when(s == 1)
        def _stage2():
            y1 = y1_scr[pl.ds(g * NB, NB)]
            sc = bn_scr[0:1].astype(jnp.bfloat16)
            sh = bn_scr[1:2].astype(jnp.bfloat16)
            a = jnp.maximum(y1 * sc + sh, jnp.bfloat16(0.0))
            _stage_store(_conv_groups(a, wb_scr), 1, 2, y2_scr)

        @pl.when(jnp.logical_and(s == 2, g == 0))
        def _fold2():
            _fold(2, g2_ref, be2_ref, 2)

        @pl.when(s == 2)
        def _epilogue():
            y2 = y2_scr[pl.ds(g * NB, NB)].astype(jnp.float32)
            a2 = jnp.maximum(y2 * bn_scr[2:3] + bn_scr[3:4], 0.0)
            xf = xb_scr[pl.ds(g * NB, NB)].reshape(M, L)
            res = jnp.concatenate(
                [jnp.dot(xf[:, k * GW:(k + 1) * GW], br256_scr[...],
                         preferred_element_type=jnp.float32)
                 for k in range(L // GW)], axis=1)
            res = res + vt_scr[2:3]
            val = jnp.maximum(a2 + res.reshape(NB, D, L), 0.0)
            valp = jnp.dot(val.reshape(M, L).astype(jnp.bfloat16), pt_scr[...],
                           preferred_element_type=jnp.float32)
            valp3 = valp.reshape(NB, D, L)
            for c in range(C):
                o_ref[:, c] = valp3[:, :, c * HW:(c + 1) * HW]

    vspec = pl.BlockSpec((1, C), lambda s, g: (0, 0))
    outf = pl.pallas_call(
        body,
        out_shape=jax.ShapeDtypeStruct((N, C, D, HW), jnp.float32),
        grid=(3, G),
        in_specs=[
            pl.BlockSpec((NB, C, D, HW),
                         lambda s, g: (jnp.where(s == 0, g, 0), 0, 0, 0)),
            pl.BlockSpec((27, C, C), lambda s, g: (0, 0, 0)),
            pl.BlockSpec((27, C, C), lambda s, g: (0, 0, 0)),
            pl.BlockSpec((C, C), lambda s, g: (0, 0)),
            vspec, vspec, vspec, vspec, vspec, vspec, vspec,
        ],
        out_specs=pl.BlockSpec((NB, C, D, HW),
                               lambda s, g: (jnp.where(s == 2, g, 0), 0, 0, 0)),
        scratch_shapes=[
            pltpu.VMEM((N, D, L), jnp.bfloat16),
            pltpu.VMEM((N, D, L), jnp.bfloat16),
            pltpu.VMEM((N, D, L), jnp.bfloat16),
            pltpu.VMEM((NG, 3 * KH * WC, GW), jnp.bfloat16),
            pltpu.VMEM((NG, 3 * KH * WC, GW), jnp.bfloat16),
            pltpu.VMEM((GW, GW), jnp.bfloat16),
            pltpu.VMEM((L, L), jnp.bfloat16),
            pltpu.VMEM((L, L), jnp.bfloat16),
            pltpu.VMEM((8, L), jnp.float32),
            pltpu.VMEM((4, L), jnp.float32),
            pltpu.VMEM((4, L), jnp.float32),
        ],
        compiler_params=pltpu.CompilerParams(
            dimension_semantics=("arbitrary", "arbitrary"),
            vmem_limit_bytes=60 * 1024 * 1024,
        ),
    )(x4, w1r, w2r, w1x1, *vecs)

    return outf.reshape(N, C, D, H, W)
